# baseline (speedup 1.0000x reference)
import functools

import jax
import jax.numpy as jnp
from jax import lax
from jax.experimental import pallas as pl
from jax.experimental.pallas import tpu as pltpu
from jax.experimental.pallas import tpu_sc as plsc

_B = 16384
_C = 100
_CW = _C // 2
_V = 1000000
_NC = 2
_NS = 16
_NW = _NC * _NS
_BPW = _B // _NW
_UNROLL = 16
_CHUNKS = _BPW // _UNROLL

_TBLK = 16384


def _tc_transpose(conf_t):
    def body(src_ref, dst_ref):
        t = src_ref[...].T
        a = t[:, :_CW].astype(jnp.bfloat16).astype(jnp.float32)
        b = t[:, _CW:].astype(jnp.bfloat16).astype(jnp.float32)
        ai = lax.bitcast_convert_type(a, jnp.int32)
        bi = lax.bitcast_convert_type(b, jnp.int32)
        packed = lax.bitwise_or(
            lax.shift_right_logical(ai, 16),
            lax.bitwise_and(bi, jnp.int32(-65536)),
        )
        dst_ref[...] = lax.bitcast_convert_type(packed, jnp.float32)

    return pl.pallas_call(
        body,
        grid=(pl.cdiv(_V, _TBLK),),
        in_specs=[pl.BlockSpec((_C, _TBLK), lambda i: (0, i))],
        out_specs=pl.BlockSpec((_TBLK, _CW), lambda i: (i, 0)),
        out_shape=jax.ShapeDtypeStruct((_V, _CW), jnp.float32),
    )(conf_t)


def _sc_gather(idx, conf):
    mesh = plsc.VectorSubcoreMesh(core_axis_name="c", subcore_axis_name="s")

    @functools.partial(
        pl.kernel,
        mesh=mesh,
        out_type=jax.ShapeDtypeStruct((_B, _CW), jnp.float32),
        scratch_types=[
            pltpu.VMEM((_BPW,), jnp.int32),
            pltpu.VMEM((_BPW, _CW), jnp.float32),
            pltpu.SemaphoreType.DMA,
        ],
    )
    def sc_kernel(idx_hbm, conf_hbm, out_hbm, idx_v, rows_v, sem):
        wid = lax.axis_index("s") * _NC + lax.axis_index("c")
        base = wid * _BPW
        pltpu.sync_copy(idx_hbm.at[pl.ds(base, _BPW)], idx_v)

        def chunk_body(chunk, _):
            off = pl.multiple_of(chunk * _UNROLL, _UNROLL)
            vv = idx_v[pl.ds(off, _UNROLL)]
            for k in range(_UNROLL):
                i = off + k
                r = vv[k]
                pltpu.async_copy(
                    conf_hbm.at[pl.ds(r, 1)],
                    rows_v.at[pl.ds(i, 1)],
                    sem,
                )
            return ()

        lax.fori_loop(0, _CHUNKS, chunk_body, ())
        pltpu.make_async_copy(conf_hbm.at[pl.ds(0, _BPW)], rows_v, sem).wait()
        pltpu.sync_copy(rows_v, out_hbm.at[pl.ds(base, _BPW)])

    return sc_kernel(idx, conf)


_BLK = 2048


def _tc_loss(x_t, gathered):
    def body(x_ref, g_ref, acc_ref):
        i = pl.program_id(0)
        x = x_ref[...]
        m = jnp.max(x, axis=0, keepdims=True)
        lse = jnp.log(jnp.sum(jnp.exp(x - m), axis=0, keepdims=True)) + m
        logsm = x - lse
        pi = lax.bitcast_convert_type(g_ref[...], jnp.int32)
        glo = lax.bitcast_convert_type(
            lax.shift_left(pi, 16), jnp.float32)
        ghi = lax.bitcast_convert_type(
            lax.bitwise_and(pi, jnp.int32(-65536)), jnp.float32)
        part = (jnp.sum(logsm[:_CW, :] * glo.T)
                + jnp.sum(logsm[_CW:, :] * ghi.T))

        @pl.when(i == 0)
        def _init():
            acc_ref[0, 0] = 0.0

        acc_ref[0, 0] += part

    acc = pl.pallas_call(
        body,
        grid=(_B // _BLK,),
        in_specs=[
            pl.BlockSpec((_C, _BLK), lambda i: (0, i)),
            pl.BlockSpec((_BLK, _CW), lambda i: (i, 0)),
        ],
        out_specs=pl.BlockSpec(memory_space=pltpu.SMEM),
        out_shape=jax.ShapeDtypeStruct((1, 1), jnp.float32),
    )(x_t, gathered)
    return acc[0, 0]


def kernel(outputs, index, confidence):
    conf_l = _tc_transpose(confidence.T)
    gathered = _sc_gather(index, conf_l)
    total = _tc_loss(outputs.T, gathered)
    return -total / _B


# speedup vs baseline: 1.0122x; 1.0122x over previous
import functools

import jax
import jax.numpy as jnp
from jax import lax
from jax.experimental import pallas as pl
from jax.experimental.pallas import tpu as pltpu
from jax.experimental.pallas import tpu_sc as plsc

_B = 16384
_C = 100
_CW = _C // 2
_V = 1000000
_NC = 2
_NS = 16
_NW = _NC * _NS
_BPW = _B // _NW
_UNROLL = 16
_CHUNKS = _BPW // _UNROLL

_TBLK = 32768


def _tc_transpose(conf_t):
    def body(src_ref, dst_ref):
        t = src_ref[...]
        a = t[:_CW, :].astype(jnp.bfloat16).astype(jnp.float32)
        b = t[_CW:, :].astype(jnp.bfloat16).astype(jnp.float32)
        ai = lax.bitcast_convert_type(a, jnp.int32)
        bi = lax.bitcast_convert_type(b, jnp.int32)
        packed = lax.bitwise_or(
            lax.shift_right_logical(ai, 16),
            lax.bitwise_and(bi, jnp.int32(-65536)),
        )
        dst_ref[...] = lax.bitcast_convert_type(packed, jnp.float32).T

    return pl.pallas_call(
        body,
        grid=(pl.cdiv(_V, _TBLK),),
        in_specs=[pl.BlockSpec((_C, _TBLK), lambda i: (0, i))],
        out_specs=pl.BlockSpec((_TBLK, _CW), lambda i: (i, 0)),
        out_shape=jax.ShapeDtypeStruct((_V, _CW), jnp.float32),
    )(conf_t)


def _sc_gather(idx, conf):
    mesh = plsc.VectorSubcoreMesh(core_axis_name="c", subcore_axis_name="s")

    @functools.partial(
        pl.kernel,
        mesh=mesh,
        out_type=jax.ShapeDtypeStruct((_B, _CW), jnp.float32),
        scratch_types=[
            pltpu.VMEM((_BPW,), jnp.int32),
            pltpu.VMEM((_BPW, _CW), jnp.float32),
            pltpu.SemaphoreType.DMA,
        ],
    )
    def sc_kernel(idx_hbm, conf_hbm, out_hbm, idx_v, rows_v, sem):
        wid = lax.axis_index("s") * _NC + lax.axis_index("c")
        base = wid * _BPW
        pltpu.sync_copy(idx_hbm.at[pl.ds(base, _BPW)], idx_v)

        def chunk_body(chunk, _):
            off = pl.multiple_of(chunk * _UNROLL, _UNROLL)
            vv = idx_v[pl.ds(off, _UNROLL)]
            for k in range(_UNROLL):
                i = off + k
                r = vv[k]
                pltpu.async_copy(
                    conf_hbm.at[pl.ds(r, 1)],
                    rows_v.at[pl.ds(i, 1)],
                    sem,
                )
            return ()

        lax.fori_loop(0, _CHUNKS, chunk_body, ())
        pltpu.make_async_copy(conf_hbm.at[pl.ds(0, _BPW)], rows_v, sem).wait()
        pltpu.sync_copy(rows_v, out_hbm.at[pl.ds(base, _BPW)])

    return sc_kernel(idx, conf)


_BLK = 2048


def _tc_loss(x_t, gathered):
    def body(x_ref, g_ref, acc_ref):
        i = pl.program_id(0)
        x = x_ref[...]
        m = jnp.max(x, axis=0, keepdims=True)
        lse = jnp.log(jnp.sum(jnp.exp(x - m), axis=0, keepdims=True)) + m
        logsm = x - lse
        pi = lax.bitcast_convert_type(g_ref[...], jnp.int32)
        glo = lax.bitcast_convert_type(
            lax.shift_left(pi, 16), jnp.float32)
        ghi = lax.bitcast_convert_type(
            lax.bitwise_and(pi, jnp.int32(-65536)), jnp.float32)
        part = (jnp.sum(logsm[:_CW, :] * glo.T)
                + jnp.sum(logsm[_CW:, :] * ghi.T))

        @pl.when(i == 0)
        def _init():
            acc_ref[0, 0] = 0.0

        acc_ref[0, 0] += part

    acc = pl.pallas_call(
        body,
        grid=(_B // _BLK,),
        in_specs=[
            pl.BlockSpec((_C, _BLK), lambda i: (0, i)),
            pl.BlockSpec((_BLK, _CW), lambda i: (i, 0)),
        ],
        out_specs=pl.BlockSpec(memory_space=pltpu.SMEM),
        out_shape=jax.ShapeDtypeStruct((1, 1), jnp.float32),
    )(x_t, gathered)
    return acc[0, 0]


def kernel(outputs, index, confidence):
    conf_l = _tc_transpose(confidence.T)
    gathered = _sc_gather(index, conf_l)
    total = _tc_loss(outputs.T, gathered)
    return -total / _B


# speedup vs baseline: 1.0131x; 1.0008x over previous
import functools

import jax
import jax.numpy as jnp
from jax import lax
from jax.experimental import pallas as pl
from jax.experimental.pallas import tpu as pltpu
from jax.experimental.pallas import tpu_sc as plsc

_B = 16384
_C = 100
_CW = _C // 2
_V = 1000000
_NC = 2
_NS = 16
_NW = _NC * _NS
_BPW = _B // _NW
_UNROLL = 16
_CHUNKS = _BPW // _UNROLL

_TBLK = 32768


def _tc_transpose(conf_t):
    def body(src_ref, dst_ref):
        t = src_ref[...]
        a = t[:_CW, :].astype(jnp.bfloat16).astype(jnp.float32)
        b = t[_CW:, :].astype(jnp.bfloat16).astype(jnp.float32)
        ai = lax.bitcast_convert_type(a, jnp.int32)
        bi = lax.bitcast_convert_type(b, jnp.int32)
        packed = lax.bitwise_or(
            lax.shift_right_logical(ai, 16),
            lax.bitwise_and(bi, jnp.int32(-65536)),
        )
        dst_ref[...] = lax.bitcast_convert_type(packed, jnp.float32).T

    return pl.pallas_call(
        body,
        grid=(pl.cdiv(_V, _TBLK),),
        in_specs=[pl.BlockSpec((_C, _TBLK), lambda i: (0, i))],
        out_specs=pl.BlockSpec((_TBLK, _CW), lambda i: (i, 0)),
        out_shape=jax.ShapeDtypeStruct((_V, _CW), jnp.float32),
    )(conf_t)


def _sc_gather(idx, conf):
    mesh = plsc.VectorSubcoreMesh(core_axis_name="c", subcore_axis_name="s")

    @functools.partial(
        pl.kernel,
        mesh=mesh,
        out_type=jax.ShapeDtypeStruct((_B, _CW), jnp.float32),
        scratch_types=[
            pltpu.VMEM((_BPW,), jnp.int32),
            pltpu.VMEM((_BPW, _CW), jnp.float32),
            pltpu.SemaphoreType.DMA,
        ],
    )
    def sc_kernel(idx_hbm, conf_hbm, out_hbm, idx_v, rows_v, sem):
        wid = lax.axis_index("s") * _NC + lax.axis_index("c")
        base = wid * _BPW
        pltpu.sync_copy(idx_hbm.at[pl.ds(base, _BPW)], idx_v)

        def chunk_body(chunk, _):
            off = pl.multiple_of(chunk * _UNROLL, _UNROLL)
            vv = idx_v[pl.ds(off, _UNROLL)]
            for k in range(_UNROLL):
                i = off + k
                r = vv[k]
                pltpu.async_copy(
                    conf_hbm.at[pl.ds(r, 1)],
                    rows_v.at[pl.ds(i, 1)],
                    sem,
                )
            return ()

        lax.fori_loop(0, _CHUNKS, chunk_body, ())
        pltpu.make_async_copy(conf_hbm.at[pl.ds(0, _BPW)], rows_v, sem).wait()
        pltpu.sync_copy(rows_v, out_hbm.at[pl.ds(base, _BPW)])

    return sc_kernel(idx, conf)


_BLK = 16384


def _tc_loss(x_t, gathered):
    def body(x_ref, g_ref, acc_ref):
        i = pl.program_id(0)
        x = x_ref[...]
        m = jnp.max(x, axis=0, keepdims=True)
        lse = jnp.log(jnp.sum(jnp.exp(x - m), axis=0, keepdims=True)) + m
        logsm = x - lse
        pi = lax.bitcast_convert_type(g_ref[...], jnp.int32)
        glo = lax.bitcast_convert_type(
            lax.shift_left(pi, 16), jnp.float32)
        ghi = lax.bitcast_convert_type(
            lax.bitwise_and(pi, jnp.int32(-65536)), jnp.float32)
        part = (jnp.sum(logsm[:_CW, :] * glo.T)
                + jnp.sum(logsm[_CW:, :] * ghi.T))

        @pl.when(i == 0)
        def _init():
            acc_ref[0, 0] = 0.0

        acc_ref[0, 0] += part

    acc = pl.pallas_call(
        body,
        grid=(_B // _BLK,),
        in_specs=[
            pl.BlockSpec((_C, _BLK), lambda i: (0, i)),
            pl.BlockSpec((_BLK, _CW), lambda i: (i, 0)),
        ],
        out_specs=pl.BlockSpec(memory_space=pltpu.SMEM),
        out_shape=jax.ShapeDtypeStruct((1, 1), jnp.float32),
    )(x_t, gathered)
    return acc[0, 0]


def kernel(outputs, index, confidence):
    conf_l = _tc_transpose(confidence.T)
    gathered = _sc_gather(index, conf_l)
    total = _tc_loss(outputs.T, gathered)
    return -total / _B
